# baseline (speedup 1.0000x reference)
import functools

import jax
import jax.numpy as jnp
from jax import lax
from jax.experimental import pallas as pl
from jax.experimental.pallas import tpu as pltpu
from jax.experimental.pallas import tpu_sc as plsc

_B, _SEQ, _N, _E = 32, 12, 325, 2600
_GCN = 64
_HID = 512
_G = _B * _SEQ
_NP = 336
_EP = 2608
_NW = 32
_GPW = _G // _NW
_NFLAT = _GPW * _NP
_EFLAT = _GPW * _EP


def _sc_body(x_hbm, src_hbm, dst_hbm, ed_hbm, a_hbm, d_hbm,
             xv, srcv, dstv, edv, av, dv):
    c = lax.axis_index("c")
    s = lax.axis_index("s")
    wid = s * 2 + c
    nb = wid * _NFLAT
    eb = wid * _EFLAT
    pltpu.sync_copy(x_hbm.at[pl.ds(nb, _NFLAT)], xv)
    pltpu.sync_copy(src_hbm.at[pl.ds(eb, _EFLAT)], srcv)
    pltpu.sync_copy(dst_hbm.at[pl.ds(eb, _EFLAT)], dstv)
    pltpu.sync_copy(ed_hbm.at[pl.ds(eb, _EFLAT)], edv)

    zeros16 = jnp.zeros((16,), jnp.float32)

    def zero(i, _):
        av[pl.ds(i * 16, 16)] = zeros16
        dv[pl.ds(i * 16, 16)] = zeros16
        return 0

    lax.fori_loop(0, _NFLAT // 16, zero, 0)

    def graph(g, _):
        base = g * _NP

        def chunk(cc, _):
            off = g * _EP + cc * 16
            sidx = srcv[pl.ds(off, 16)] + base
            didx = dstv[pl.ds(off, 16)] + base
            w = edv[pl.ds(off, 16)]
            xg = plsc.load_gather(xv, [sidx])
            plsc.addupdate_scatter(av, [didx], xg * w)
            plsc.addupdate_scatter(dv, [didx], w)
            return 0

        lax.fori_loop(0, _EP // 16, chunk, 0)
        return 0

    lax.fori_loop(0, _GPW, graph, 0)

    def out_copy(g, _):
        off = (g * _B + wid) * _NP
        pltpu.sync_copy(av.at[pl.ds(g * _NP, _NP)], a_hbm.at[pl.ds(off, _NP)])
        pltpu.sync_copy(dv.at[pl.ds(g * _NP, _NP)], d_hbm.at[pl.ds(off, _NP)])
        return 0

    lax.fori_loop(0, _GPW, out_copy, 0)


def _sc_segment_sums(xf, srcf, dstf, edf):
    mesh = plsc.VectorSubcoreMesh(core_axis_name="c", subcore_axis_name="s")
    f32 = jnp.float32
    out = jax.ShapeDtypeStruct((_G * _NP,), f32)
    fn = pl.kernel(
        _sc_body,
        out_type=[out, out],
        mesh=mesh,
        scratch_types=[
            pltpu.VMEM((_NFLAT,), f32),
            pltpu.VMEM((_EFLAT,), jnp.int32),
            pltpu.VMEM((_EFLAT,), jnp.int32),
            pltpu.VMEM((_EFLAT,), f32),
            pltpu.VMEM((_NFLAT,), f32),
            pltpu.VMEM((_NFLAT,), f32),
        ],
        compiler_params=pltpu.CompilerParams(needs_layout_passes=False),
    )
    return fn(xf, srcf, dstf, edf)


_KIN = _N * _GCN
_NBLK = 256
_KB = 1600


def _tc1_body(a_ref, d_ref, wgt_ref, bgt_ref, w_ref, bih0_ref, z_ref, g_ref):
    n = pl.program_id(0)

    @pl.when(n == 0)
    def _():
        ad = jnp.concatenate([a_ref[...], d_ref[...]], axis=0)
        for kb in range(_KIN // _KB):
            rows = lax.broadcasted_iota(jnp.int32, (_N, _KB), 0)
            cols = lax.broadcasted_iota(jnp.int32, (_N, _KB), 1)
            ek = (rows == kb * (_KB // _GCN) + cols // _GCN)
            ek = ek.astype(jnp.float32)
            adb = lax.dot_general(ad, ek, (((1,), (0,)), ((), ())),
                                  preferred_element_type=jnp.float32)
            sl = pl.ds(kb * _KB, _KB)
            wgv = wgt_ref[0, sl]
            bgv = bgt_ref[0, sl]
            pre = adb[:_G, :] * wgv[None, :] + adb[_G:, :] * bgv[None, :]
            g_ref[:, sl] = jnp.tanh(jnp.maximum(pre, 0.0)) \
                .astype(jnp.bfloat16)

    z_ref[...] = lax.dot_general(
        g_ref[...], w_ref[...], (((1,), (1,)), ((), ())),
        preferred_element_type=jnp.float32) + bih0_ref[...]


def _tc1(a, d, wgt, bgt, w, bih0):
    return pl.pallas_call(
        _tc1_body,
        grid=(4 * _HID // _NBLK,),
        in_specs=[
            pl.BlockSpec((_G, _N), lambda n: (0, 0)),
            pl.BlockSpec((_G, _N), lambda n: (0, 0)),
            pl.BlockSpec((1, _KIN), lambda n: (0, 0)),
            pl.BlockSpec((1, _KIN), lambda n: (0, 0)),
            pl.BlockSpec((_NBLK, _KIN), lambda n: (n, 0)),
            pl.BlockSpec((1, _NBLK), lambda n: (0, n)),
        ],
        out_specs=pl.BlockSpec((_G, _NBLK), lambda n: (0, n)),
        out_shape=jax.ShapeDtypeStruct((_G, 4 * _HID), jnp.float32),
        scratch_shapes=[pltpu.VMEM((_G, _KIN), jnp.bfloat16)],
        compiler_params=pltpu.CompilerParams(
            dimension_semantics=("arbitrary",),
            vmem_limit_bytes=100 * 1024 * 1024),
    )(a, d, wgt, bgt, w, bih0)


def _dot_t(x, w):
    return lax.dot_general(
        x, w, (((1,), (1,)), ((), ())), preferred_element_type=jnp.float32)


def _tc2_body(z_ref, whh0_ref, bhh0_ref, wih1_ref, whh1_ref, b1_ref,
              wfc_ref, bfc_ref, out_ref, h0_ref, c0_ref, h1_ref, c1_ref):
    zero_h = jnp.zeros((_B, _HID), jnp.float32)
    h0_ref[...] = zero_h
    c0_ref[...] = zero_h
    h1_ref[...] = zero_h
    c1_ref[...] = zero_h

    def step(t, _):
        x_t = z_ref[pl.ds(t * _B, _B), :]
        g0 = x_t + _dot_t(h0_ref[...], whh0_ref[...]) + bhh0_ref[...]
        i0 = jax.nn.sigmoid(g0[:, :_HID])
        f0 = jax.nn.sigmoid(g0[:, _HID:2 * _HID])
        gg0 = jnp.tanh(g0[:, 2 * _HID:3 * _HID])
        o0 = jax.nn.sigmoid(g0[:, 3 * _HID:])
        c0 = f0 * c0_ref[...] + i0 * gg0
        h0 = o0 * jnp.tanh(c0)
        c0_ref[...] = c0
        h0_ref[...] = h0

        g1 = _dot_t(h0, wih1_ref[...]) + _dot_t(h1_ref[...], whh1_ref[...]) \
            + b1_ref[...]
        i1 = jax.nn.sigmoid(g1[:, :_HID])
        f1 = jax.nn.sigmoid(g1[:, _HID:2 * _HID])
        gg1 = jnp.tanh(g1[:, 2 * _HID:3 * _HID])
        o1 = jax.nn.sigmoid(g1[:, 3 * _HID:])
        c1 = f1 * c1_ref[...] + i1 * gg1
        c1_ref[...] = c1
        h1_ref[...] = o1 * jnp.tanh(c1)
        return 0

    lax.fori_loop(0, _SEQ, step, 0)

    h = jnp.tanh(h1_ref[...])
    out_ref[...] = _dot_t(h, wfc_ref[...]) + bfc_ref[...]


def _tc2(z0s, whh0, bhh0, wih1, whh1, b1, wfc, bfc):
    nout = wfc.shape[0]
    return pl.pallas_call(
        _tc2_body,
        out_shape=jax.ShapeDtypeStruct((_B, nout), jnp.float32),
        scratch_shapes=[pltpu.VMEM((_B, _HID), jnp.float32)] * 4,
    )(z0s, whh0, bhh0, wih1, whh1, b1, wfc, bfc)


def kernel(x_sequences, edge_indices_sequences, edge_distances_sequences,
           Wg, bg, Wih0, Whh0, bih0, bhh0, Wih1, Whh1, bih1, bhh1, Wfc, bfc):
    f32 = jnp.float32

    x2 = x_sequences.reshape(_G, _N)
    xp = jnp.pad(x2, ((0, 0), (0, _NP - _N)))
    ei = edge_indices_sequences.reshape(_G, 2, _E)
    src = jnp.pad(ei[:, 0, :], ((0, 0), (0, _EP - _E)))
    dst = jnp.pad(ei[:, 1, :], ((0, 0), (0, _EP - _E)))
    ed = jnp.pad(edge_distances_sequences.reshape(_G, _E),
                 ((0, 0), (0, _EP - _E)))
    af, df = _sc_segment_sums(xp.reshape(-1), src.reshape(-1),
                              dst.reshape(-1), ed.reshape(-1))
    a = af.reshape(_G, _NP)[:, :_N]
    d = df.reshape(_G, _NP)[:, :_N]

    wgt = jnp.tile(Wg.reshape(_GCN), _N).reshape(1, _KIN)
    bgt = jnp.tile(bg, _N).reshape(1, _KIN)
    z0 = _tc1(a, d, wgt, bgt, Wih0.astype(jnp.bfloat16),
              bih0.reshape(1, 4 * _HID))

    b1 = (bih1 + bhh1).reshape(1, 4 * _HID)
    out = _tc2(z0, Whh0, bhh0.reshape(1, 4 * _HID), Wih1, Whh1, b1,
               Wfc, bfc.reshape(1, -1))
    return out.reshape(_B, _SEQ, _N).astype(f32)


# speedup vs baseline: 1.0068x; 1.0068x over previous
import functools

import jax
import jax.numpy as jnp
from jax import lax
from jax.experimental import pallas as pl
from jax.experimental.pallas import tpu as pltpu
from jax.experimental.pallas import tpu_sc as plsc

_B, _SEQ, _N, _E = 32, 12, 325, 2600
_GCN = 64
_HID = 512
_G = _B * _SEQ
_NP = 336
_EP = 2608
_NW = 32
_GPW = _G // _NW
_NFLAT = _GPW * _NP
_EFLAT = _GPW * _EP


def _sc_body(x_hbm, src_hbm, dst_hbm, ed_hbm, a_hbm, d_hbm,
             xv, srcv, dstv, edv, av, dv):
    c = lax.axis_index("c")
    s = lax.axis_index("s")
    wid = s * 2 + c
    nb = wid * _NFLAT
    eb = wid * _EFLAT
    pltpu.sync_copy(x_hbm.at[pl.ds(nb, _NFLAT)], xv)
    pltpu.sync_copy(src_hbm.at[pl.ds(eb, _EFLAT)], srcv)
    pltpu.sync_copy(dst_hbm.at[pl.ds(eb, _EFLAT)], dstv)
    pltpu.sync_copy(ed_hbm.at[pl.ds(eb, _EFLAT)], edv)

    zeros16 = jnp.zeros((16,), jnp.float32)

    def zero(i, _):
        av[pl.ds(i * 16, 16)] = zeros16
        dv[pl.ds(i * 16, 16)] = zeros16
        return 0

    lax.fori_loop(0, _NFLAT // 16, zero, 0)

    def graph(g, _):
        base = g * _NP

        def chunk(cc, _):
            off = g * _EP + cc * 16
            sidx = srcv[pl.ds(off, 16)] + base
            didx = dstv[pl.ds(off, 16)] + base
            w = edv[pl.ds(off, 16)]
            xg = plsc.load_gather(xv, [sidx])
            plsc.addupdate_scatter(av, [didx], xg * w)
            plsc.addupdate_scatter(dv, [didx], w)
            return 0

        lax.fori_loop(0, _EP // 16, chunk, 0)
        return 0

    lax.fori_loop(0, _GPW, graph, 0)

    def out_copy(g, _):
        off = (g * _B + wid) * _NP
        pltpu.sync_copy(av.at[pl.ds(g * _NP, _NP)], a_hbm.at[pl.ds(off, _NP)])
        pltpu.sync_copy(dv.at[pl.ds(g * _NP, _NP)], d_hbm.at[pl.ds(off, _NP)])
        return 0

    lax.fori_loop(0, _GPW, out_copy, 0)


def _sc_segment_sums(xf, srcf, dstf, edf):
    mesh = plsc.VectorSubcoreMesh(core_axis_name="c", subcore_axis_name="s")
    f32 = jnp.float32
    out = jax.ShapeDtypeStruct((_G * _NP,), f32)
    fn = pl.kernel(
        _sc_body,
        out_type=[out, out],
        mesh=mesh,
        scratch_types=[
            pltpu.VMEM((_NFLAT,), f32),
            pltpu.VMEM((_EFLAT,), jnp.int32),
            pltpu.VMEM((_EFLAT,), jnp.int32),
            pltpu.VMEM((_EFLAT,), f32),
            pltpu.VMEM((_NFLAT,), f32),
            pltpu.VMEM((_NFLAT,), f32),
        ],
        compiler_params=pltpu.CompilerParams(needs_layout_passes=False),
    )
    return fn(xf, srcf, dstf, edf)


_KIN = _N * _GCN
_NBLK = 128
_KB = 1600


def _tc1_body(a_ref, d_ref, wgt_ref, bgt_ref, w_ref, bih0_ref, z_ref, g_ref):
    n = pl.program_id(0)

    @pl.when(n == 0)
    def _():
        ad = jnp.concatenate([a_ref[...], d_ref[...]], axis=0)
        for kb in range(_KIN // _KB):
            rows = lax.broadcasted_iota(jnp.int32, (_N, _KB), 0)
            cols = lax.broadcasted_iota(jnp.int32, (_N, _KB), 1)
            ek = (rows == kb * (_KB // _GCN) + cols // _GCN)
            ek = ek.astype(jnp.float32)
            adb = lax.dot_general(ad, ek, (((1,), (0,)), ((), ())),
                                  preferred_element_type=jnp.float32)
            sl = pl.ds(kb * _KB, _KB)
            wgv = wgt_ref[0, sl]
            bgv = bgt_ref[0, sl]
            pre = adb[:_G, :] * wgv[None, :] + adb[_G:, :] * bgv[None, :]
            g_ref[:, sl] = jnp.tanh(jnp.maximum(pre, 0.0)) \
                .astype(jnp.bfloat16)

    wb = w_ref[...].astype(jnp.bfloat16)
    z_ref[...] = lax.dot_general(
        g_ref[...], wb, (((1,), (1,)), ((), ())),
        preferred_element_type=jnp.float32) + bih0_ref[...]


def _tc1(a, d, wgt, bgt, w, bih0):
    return pl.pallas_call(
        _tc1_body,
        grid=(4 * _HID // _NBLK,),
        in_specs=[
            pl.BlockSpec((_G, _N), lambda n: (0, 0)),
            pl.BlockSpec((_G, _N), lambda n: (0, 0)),
            pl.BlockSpec((1, _KIN), lambda n: (0, 0)),
            pl.BlockSpec((1, _KIN), lambda n: (0, 0)),
            pl.BlockSpec((_NBLK, _KIN), lambda n: (n, 0)),
            pl.BlockSpec((1, _NBLK), lambda n: (0, n)),
        ],
        out_specs=pl.BlockSpec((_G, _NBLK), lambda n: (0, n)),
        out_shape=jax.ShapeDtypeStruct((_G, 4 * _HID), jnp.float32),
        scratch_shapes=[pltpu.VMEM((_G, _KIN), jnp.bfloat16)],
        compiler_params=pltpu.CompilerParams(
            dimension_semantics=("arbitrary",),
            vmem_limit_bytes=100 * 1024 * 1024),
    )(a, d, wgt, bgt, w, bih0)


def _dot_t(x, w):
    return lax.dot_general(
        x, w, (((1,), (1,)), ((), ())), preferred_element_type=jnp.float32)


def _tc2_body(z_ref, whh0_ref, bhh0_ref, wih1_ref, whh1_ref, b1_ref,
              wfc_ref, bfc_ref, out_ref, h0_ref, c0_ref, h1_ref, c1_ref):
    zero_h = jnp.zeros((_B, _HID), jnp.float32)
    h0_ref[...] = zero_h
    c0_ref[...] = zero_h
    h1_ref[...] = zero_h
    c1_ref[...] = zero_h

    def step(t, _):
        x_t = z_ref[pl.ds(t * _B, _B), :]
        g0 = x_t + _dot_t(h0_ref[...], whh0_ref[...]) + bhh0_ref[...]
        i0 = jax.nn.sigmoid(g0[:, :_HID])
        f0 = jax.nn.sigmoid(g0[:, _HID:2 * _HID])
        gg0 = jnp.tanh(g0[:, 2 * _HID:3 * _HID])
        o0 = jax.nn.sigmoid(g0[:, 3 * _HID:])
        c0 = f0 * c0_ref[...] + i0 * gg0
        h0 = o0 * jnp.tanh(c0)
        c0_ref[...] = c0
        h0_ref[...] = h0

        g1 = _dot_t(h0, wih1_ref[...]) + _dot_t(h1_ref[...], whh1_ref[...]) \
            + b1_ref[...]
        i1 = jax.nn.sigmoid(g1[:, :_HID])
        f1 = jax.nn.sigmoid(g1[:, _HID:2 * _HID])
        gg1 = jnp.tanh(g1[:, 2 * _HID:3 * _HID])
        o1 = jax.nn.sigmoid(g1[:, 3 * _HID:])
        c1 = f1 * c1_ref[...] + i1 * gg1
        c1_ref[...] = c1
        h1_ref[...] = o1 * jnp.tanh(c1)
        return 0

    lax.fori_loop(0, _SEQ, step, 0)

    h = jnp.tanh(h1_ref[...])
    out_ref[...] = _dot_t(h, wfc_ref[...]) + bfc_ref[...]


def _tc2(z0s, whh0, bhh0, wih1, whh1, b1, wfc, bfc):
    nout = wfc.shape[0]
    return pl.pallas_call(
        _tc2_body,
        out_shape=jax.ShapeDtypeStruct((_B, nout), jnp.float32),
        scratch_shapes=[pltpu.VMEM((_B, _HID), jnp.float32)] * 4,
    )(z0s, whh0, bhh0, wih1, whh1, b1, wfc, bfc)


def kernel(x_sequences, edge_indices_sequences, edge_distances_sequences,
           Wg, bg, Wih0, Whh0, bih0, bhh0, Wih1, Whh1, bih1, bhh1, Wfc, bfc):
    f32 = jnp.float32

    x2 = x_sequences.reshape(_G, _N)
    xp = jnp.pad(x2, ((0, 0), (0, _NP - _N)))
    ei = edge_indices_sequences.reshape(_G, 2, _E)
    src = jnp.pad(ei[:, 0, :], ((0, 0), (0, _EP - _E)))
    dst = jnp.pad(ei[:, 1, :], ((0, 0), (0, _EP - _E)))
    ed = jnp.pad(edge_distances_sequences.reshape(_G, _E),
                 ((0, 0), (0, _EP - _E)))
    af, df = _sc_segment_sums(xp.reshape(-1), src.reshape(-1),
                              dst.reshape(-1), ed.reshape(-1))
    a = af.reshape(_G, _NP)[:, :_N]
    d = df.reshape(_G, _NP)[:, :_N]

    wgt = jnp.tile(Wg.reshape(_GCN), _N).reshape(1, _KIN)
    bgt = jnp.tile(bg, _N).reshape(1, _KIN)
    z0 = _tc1(a, d, wgt, bgt, Wih0, bih0.reshape(1, 4 * _HID))

    b1 = (bih1 + bhh1).reshape(1, 4 * _HID)
    out = _tc2(z0, Whh0, bhh0.reshape(1, 4 * _HID), Wih1, Whh1, b1,
               Wfc, bfc.reshape(1, -1))
    return out.reshape(_B, _SEQ, _N).astype(f32)


# speedup vs baseline: 1.0473x; 1.0402x over previous
import functools

import jax
import jax.numpy as jnp
from jax import lax
from jax.experimental import pallas as pl
from jax.experimental.pallas import tpu as pltpu
from jax.experimental.pallas import tpu_sc as plsc

_B, _SEQ, _N, _E = 32, 12, 325, 2600
_GCN = 64
_HID = 512
_G = _B * _SEQ
_NP = 336
_NW = 32
_GPW = _G // _NW
_NFLAT = _GPW * _NP
_XPW = _GPW * _N
_XBUF = _XPW + 4
_EIPW = _GPW * 2 * _E
_EDPW = _GPW * _E
_EFULL = _E // 16


def _sc_body(x_hbm, ei_hbm, ed_hbm, a_hbm, d_hbm,
             xv, eiv, edv, av, dv):
    c = lax.axis_index("c")
    s = lax.axis_index("s")
    wid = s * 2 + c
    xoff = (wid % 2) * 4
    xstart = pl.multiple_of(wid * _XPW - xoff, 8)
    pltpu.sync_copy(x_hbm.at[pl.ds(xstart, _XBUF)], xv)
    pltpu.sync_copy(ei_hbm.at[pl.ds(wid * _EIPW, _EIPW)],
                    eiv.at[pl.ds(0, _EIPW)])
    pltpu.sync_copy(ed_hbm.at[pl.ds(wid * _EDPW, _EDPW)],
                    edv.at[pl.ds(0, _EDPW)])

    zeros16 = jnp.zeros((16,), jnp.float32)

    def zero(i, _):
        av[pl.ds(i * 16, 16)] = zeros16
        dv[pl.ds(i * 16, 16)] = zeros16
        return 0

    lax.fori_loop(0, _NFLAT // 16, zero, 0)

    tail_mask = lax.iota(jnp.int32, 16) < (_E - _EFULL * 16)

    def graph(g, _):
        base = g * _NP
        xbase = xoff + g * _N
        soff = g * 2 * _E
        doff = soff + _E
        woff = g * _E

        def chunk(cc, _):
            sidx = eiv[pl.ds(soff + cc * 16, 16)] + xbase
            didx = eiv[pl.ds(doff + cc * 16, 16)] + base
            w = edv[pl.ds(woff + cc * 16, 16)]
            xg = plsc.load_gather(xv, [sidx])
            plsc.addupdate_scatter(av, [didx], xg * w)
            plsc.addupdate_scatter(dv, [didx], w)
            return 0

        lax.fori_loop(0, _EFULL, chunk, 0)

        te = _EFULL * 16
        sidx = eiv[pl.ds(soff + te, 16)] + xbase
        didx = eiv[pl.ds(doff + te, 16)] + base
        w = edv[pl.ds(woff + te, 16)]
        xg = plsc.load_gather(xv, [sidx], mask=tail_mask)
        plsc.addupdate_scatter(av, [didx], xg * w, mask=tail_mask)
        plsc.addupdate_scatter(dv, [didx], w, mask=tail_mask)
        return 0

    lax.fori_loop(0, _GPW, graph, 0)

    def out_copy(g, _):
        off = (g * _B + wid) * _NP
        pltpu.sync_copy(av.at[pl.ds(g * _NP, _NP)], a_hbm.at[pl.ds(off, _NP)])
        pltpu.sync_copy(dv.at[pl.ds(g * _NP, _NP)], d_hbm.at[pl.ds(off, _NP)])
        return 0

    lax.fori_loop(0, _GPW, out_copy, 0)


def _sc_segment_sums(xf, eif, edf):
    mesh = plsc.VectorSubcoreMesh(core_axis_name="c", subcore_axis_name="s")
    f32 = jnp.float32
    out = jax.ShapeDtypeStruct((_G * _NP,), f32)
    fn = pl.kernel(
        _sc_body,
        out_type=[out, out],
        mesh=mesh,
        scratch_types=[
            pltpu.VMEM((_XBUF,), f32),
            pltpu.VMEM((_EIPW + 16,), jnp.int32),
            pltpu.VMEM((_EDPW + 16,), f32),
            pltpu.VMEM((_NFLAT,), f32),
            pltpu.VMEM((_NFLAT,), f32),
        ],
        compiler_params=pltpu.CompilerParams(needs_layout_passes=False),
    )
    return fn(xf, eif, edf)


_KIN = _N * _GCN
_NBLK = 128
_KB = 1600


def _tc1_body(a_ref, d_ref, wgt_ref, bgt_ref, w_ref, bih0_ref, z_ref, g_ref):
    n = pl.program_id(0)

    @pl.when(n == 0)
    def _():
        ad = jnp.concatenate([a_ref[...], d_ref[...]], axis=0)
        for kb in range(_KIN // _KB):
            rows = lax.broadcasted_iota(jnp.int32, (_N, _KB), 0)
            cols = lax.broadcasted_iota(jnp.int32, (_N, _KB), 1)
            ek = (rows == kb * (_KB // _GCN) + cols // _GCN)
            ek = ek.astype(jnp.float32)
            adb = lax.dot_general(ad, ek, (((1,), (0,)), ((), ())),
                                  preferred_element_type=jnp.float32)
            sl = pl.ds(kb * _KB, _KB)
            wgv = wgt_ref[0, sl]
            bgv = bgt_ref[0, sl]
            pre = adb[:_G, :] * wgv[None, :] + adb[_G:, :] * bgv[None, :]
            g_ref[:, sl] = jnp.tanh(jnp.maximum(pre, 0.0)) \
                .astype(jnp.bfloat16)

    wb = w_ref[...].astype(jnp.bfloat16)
    z_ref[...] = lax.dot_general(
        g_ref[...], wb, (((1,), (1,)), ((), ())),
        preferred_element_type=jnp.float32) + bih0_ref[...]


def _tc1(a, d, wgt, bgt, w, bih0):
    return pl.pallas_call(
        _tc1_body,
        grid=(4 * _HID // _NBLK,),
        in_specs=[
            pl.BlockSpec((_G, _N), lambda n: (0, 0)),
            pl.BlockSpec((_G, _N), lambda n: (0, 0)),
            pl.BlockSpec((1, _KIN), lambda n: (0, 0)),
            pl.BlockSpec((1, _KIN), lambda n: (0, 0)),
            pl.BlockSpec((_NBLK, _KIN), lambda n: (n, 0)),
            pl.BlockSpec((1, _NBLK), lambda n: (0, n)),
        ],
        out_specs=pl.BlockSpec((_G, _NBLK), lambda n: (0, n)),
        out_shape=jax.ShapeDtypeStruct((_G, 4 * _HID), jnp.float32),
        scratch_shapes=[pltpu.VMEM((_G, _KIN), jnp.bfloat16)],
        compiler_params=pltpu.CompilerParams(
            dimension_semantics=("arbitrary",),
            vmem_limit_bytes=100 * 1024 * 1024),
    )(a, d, wgt, bgt, w, bih0)


def _dot_t(x, w):
    return lax.dot_general(
        x, w, (((1,), (1,)), ((), ())), preferred_element_type=jnp.float32)


def _tc2_body(z_ref, whh0_ref, bhh0_ref, wih1_ref, whh1_ref, b1_ref,
              wfc_ref, bfc_ref, out_ref, h0_ref, c0_ref, h1_ref, c1_ref):
    zero_h = jnp.zeros((_B, _HID), jnp.float32)
    h0_ref[...] = zero_h
    c0_ref[...] = zero_h
    h1_ref[...] = zero_h
    c1_ref[...] = zero_h

    def step(t, _):
        x_t = z_ref[pl.ds(t * _B, _B), :]
        g0 = x_t + _dot_t(h0_ref[...], whh0_ref[...]) + bhh0_ref[...]
        i0 = jax.nn.sigmoid(g0[:, :_HID])
        f0 = jax.nn.sigmoid(g0[:, _HID:2 * _HID])
        gg0 = jnp.tanh(g0[:, 2 * _HID:3 * _HID])
        o0 = jax.nn.sigmoid(g0[:, 3 * _HID:])
        c0 = f0 * c0_ref[...] + i0 * gg0
        h0 = o0 * jnp.tanh(c0)
        c0_ref[...] = c0
        h0_ref[...] = h0

        g1 = _dot_t(h0, wih1_ref[...]) + _dot_t(h1_ref[...], whh1_ref[...]) \
            + b1_ref[...]
        i1 = jax.nn.sigmoid(g1[:, :_HID])
        f1 = jax.nn.sigmoid(g1[:, _HID:2 * _HID])
        gg1 = jnp.tanh(g1[:, 2 * _HID:3 * _HID])
        o1 = jax.nn.sigmoid(g1[:, 3 * _HID:])
        c1 = f1 * c1_ref[...] + i1 * gg1
        c1_ref[...] = c1
        h1_ref[...] = o1 * jnp.tanh(c1)
        return 0

    lax.fori_loop(0, _SEQ, step, 0)

    h = jnp.tanh(h1_ref[...])
    out_ref[...] = _dot_t(h, wfc_ref[...]) + bfc_ref[...]


def _tc2(z0s, whh0, bhh0, wih1, whh1, b1, wfc, bfc):
    nout = wfc.shape[0]
    return pl.pallas_call(
        _tc2_body,
        out_shape=jax.ShapeDtypeStruct((_B, nout), jnp.float32),
        scratch_shapes=[pltpu.VMEM((_B, _HID), jnp.float32)] * 4,
    )(z0s, whh0, bhh0, wih1, whh1, b1, wfc, bfc)


def kernel(x_sequences, edge_indices_sequences, edge_distances_sequences,
           Wg, bg, Wih0, Whh0, bih0, bhh0, Wih1, Whh1, bih1, bhh1, Wfc, bfc):
    f32 = jnp.float32

    af, df = _sc_segment_sums(x_sequences.reshape(-1),
                              edge_indices_sequences.reshape(-1),
                              edge_distances_sequences.reshape(-1))
    a = af.reshape(_G, _NP)[:, :_N]
    d = df.reshape(_G, _NP)[:, :_N]

    wgt = jnp.tile(Wg.reshape(_GCN), _N).reshape(1, _KIN)
    bgt = jnp.tile(bg, _N).reshape(1, _KIN)
    z0 = _tc1(a, d, wgt, bgt, Wih0, bih0.reshape(1, 4 * _HID))

    b1 = (bih1 + bhh1).reshape(1, 4 * _HID)
    out = _tc2(z0, Whh0, bhh0.reshape(1, 4 * _HID), Wih1, Whh1, b1,
               Wfc, bfc.reshape(1, -1))
    return out.reshape(_B, _SEQ, _N).astype(f32)
